# speedup vs baseline: 2.9431x; 2.9431x over previous
import jax
import jax.numpy as jnp
from jax.experimental import pallas as pl
from jax.experimental.pallas import tpu as pltpu

N_NODES = 10000
D_IN = 128
D_H = 128
D_OUT = 64
EPS = 1e-5


def _tc1_body(x_ref, w1_ref, deg_ref, h1_ref, hs_ref, self_ref, dinv_ref):
    deg = deg_ref[0, :] + 1.0
    dinv = jax.lax.rsqrt(deg)
    h1 = jnp.dot(x_ref[...], w1_ref[...], preferred_element_type=jnp.float32)
    h1_ref[...] = h1
    hs_ref[...] = h1 * dinv[:, None]
    self_ref[...] = h1 * (dinv * dinv)[:, None]
    dinv_ref[...] = dinv[:, None]


def _tc2_body(acc_ref, dinv_ref, self_ref, g_ref, bt_ref, w2_ref,
              hs2_ref, self2_ref):
    dinv = dinv_ref[...]
    out1 = acc_ref[...] * dinv + self_ref[...]
    mean = jnp.mean(out1, axis=0, keepdims=True)
    xc = out1 - mean
    var = jnp.mean(xc * xc, axis=0, keepdims=True)
    y = jnp.maximum(xc * jax.lax.rsqrt(var + EPS) * g_ref[...] + bt_ref[...], 0.0)
    h2 = jnp.dot(y, w2_ref[...], preferred_element_type=jnp.float32)
    hs2_ref[...] = h2 * dinv
    self2_ref[...] = h2 * (dinv * dinv)


def _tc3_body(acc_ref, dinv_ref, self2_ref, b2_ref, out_ref):
    out_ref[...] = acc_ref[...] * dinv_ref[...] + self2_ref[...] + b2_ref[...]


def kernel(x, edge_index, weight, W1, b1, g1, bt1, W2, b2):
    src = edge_index[0]
    dst = edge_index[1]
    w = weight

    deg0 = jnp.zeros((N_NODES,), jnp.float32).at[dst].add(w)
    deg_in = deg0.reshape(1, N_NODES)

    f32 = jnp.float32
    h1, hs1, self1, dinv = pl.pallas_call(
        _tc1_body,
        out_shape=(
            jax.ShapeDtypeStruct((N_NODES, D_H), f32),
            jax.ShapeDtypeStruct((N_NODES, D_H), f32),
            jax.ShapeDtypeStruct((N_NODES, D_H), f32),
            jax.ShapeDtypeStruct((N_NODES, 1), f32),
        ),
    )(x, W1, deg_in)

    acc1 = jnp.zeros((N_NODES, D_H), f32).at[dst].add(w[:, None] * hs1[src])

    hs2, self2 = pl.pallas_call(
        _tc2_body,
        out_shape=(
            jax.ShapeDtypeStruct((N_NODES, D_OUT), f32),
            jax.ShapeDtypeStruct((N_NODES, D_OUT), f32),
        ),
    )(acc1, dinv, self1, g1.reshape(1, D_H), bt1.reshape(1, D_H), W2)

    acc2 = jnp.zeros((N_NODES, D_OUT), f32).at[dst].add(w[:, None] * hs2[src])

    out = pl.pallas_call(
        _tc3_body,
        out_shape=jax.ShapeDtypeStruct((N_NODES, D_OUT), f32),
    )(acc2, dinv, self2, b2.reshape(1, D_OUT))
    return out


# speedup vs baseline: 9.7684x; 3.3191x over previous
import functools

import jax
import jax.numpy as jnp
from jax import lax
from jax.experimental import pallas as pl
from jax.experimental.pallas import tpu as pltpu
from jax.experimental.pallas import tpu_sc as plsc

N_NODES = 10000
N_EDGES = 320000
D_IN = 128
D_H = 128
D_OUT = 64
EPS = 1e-5

NC = 2
NS = 16
NW = NC * NS
EPT = N_EDGES // NW
NP = 10240
ROWS_PT = NP // NS

import numpy as np

_SPLAT_IDX = [np.full((16,), r, dtype=np.int32) for r in range(16)]


def _make_deg_kernel(chunk=80):
    n_chunks = EPT // chunk
    mesh = plsc.VectorSubcoreMesh(core_axis_name="c", subcore_axis_name="s")

    @functools.partial(
        pl.kernel,
        out_type=jax.ShapeDtypeStruct((NC, NP), jnp.float32),
        mesh=mesh,
        scratch_types=[
            pltpu.VMEM((chunk,), jnp.int32),
            pltpu.VMEM((chunk,), jnp.float32),
            pltpu.VMEM_SHARED((NP,), jnp.float32),
        ],
    )
    def deg_kernel(dst_hbm, w_hbm, zeros_hbm, out_hbm, dst_v, w_v, deg_sp):
        cid = lax.axis_index("c")
        sid = lax.axis_index("s")
        pltpu.sync_copy(zeros_hbm, deg_sp.at[pl.ds(sid * 640, 640)])
        plsc.subcore_barrier()
        base0 = (sid * NC + cid) * EPT

        def body(ci, carry):
            b = base0 + ci * chunk
            pltpu.sync_copy(dst_hbm.at[pl.ds(b, chunk)], dst_v)
            pltpu.sync_copy(w_hbm.at[pl.ds(b, chunk)], w_v)
            pltpu.sync_copy(w_v, deg_sp.at[dst_v], add=True)
            return carry

        lax.fori_loop(0, n_chunks, body, 0)
        plsc.subcore_barrier()
        pltpu.sync_copy(deg_sp.at[pl.ds(sid * 640, 640)],
                        out_hbm.at[cid, pl.ds(sid * 640, 640)])

    return deg_kernel


def _make_msg_kernel(D, chunk=80):
    n_chunks = EPT // chunk
    mesh = plsc.VectorSubcoreMesh(core_axis_name="c", subcore_axis_name="s")

    @functools.partial(
        pl.kernel,
        out_type=jax.ShapeDtypeStruct((NC, NP, D), jnp.float32),
        mesh=mesh,
        scratch_types=[
            pltpu.VMEM((chunk,), jnp.int32),
            pltpu.VMEM((chunk,), jnp.int32),
            pltpu.VMEM((chunk,), jnp.float32),
            pltpu.VMEM((chunk, D), jnp.float32),
            pltpu.VMEM_SHARED((NP, D), jnp.float32),
            pltpu.SemaphoreType.DMA,
        ],
    )
    def msg_kernel(hs_hbm, src_hbm, dst_hbm, w_hbm, zeros_hbm, out_hbm,
                   src_v, dst_v, w_v, rows_v, acc_sp, sem):
        cid = lax.axis_index("c")
        sid = lax.axis_index("s")
        pltpu.sync_copy(zeros_hbm, acc_sp.at[pl.ds(sid * ROWS_PT, ROWS_PT)])
        plsc.subcore_barrier()
        base0 = (sid * NC + cid) * EPT

        def body(ci, carry):
            b = base0 + ci * chunk
            pltpu.sync_copy(src_hbm.at[pl.ds(b, chunk)], src_v)
            pltpu.sync_copy(dst_hbm.at[pl.ds(b, chunk)], dst_v)
            pltpu.sync_copy(w_hbm.at[pl.ds(b, chunk)], w_v)
            pltpu.async_copy(hs_hbm.at[src_v], rows_v, sem).wait()
            for gi in range(chunk // 16):
                wv = w_v[pl.ds(gi * 16, 16)]
                for r in range(16):
                    row = gi * 16 + r
                    splat = lax.broadcast_in_dim(wv[r], (16,), ())
                    for c2 in range(D // 16):
                        sl = pl.ds(c2 * 16, 16)
                        rows_v[row, sl] = rows_v[row, sl] * splat
            pltpu.sync_copy(rows_v, acc_sp.at[dst_v], add=True)
            return carry

        lax.fori_loop(0, n_chunks, body, 0)
        plsc.subcore_barrier()
        r0 = sid * ROWS_PT
        pltpu.sync_copy(acc_sp.at[pl.ds(r0, ROWS_PT)],
                        out_hbm.at[cid, pl.ds(r0, ROWS_PT)])

    return msg_kernel


_deg = _make_deg_kernel()
_msg128 = _make_msg_kernel(D_H)


def _tc1_body(x_ref, w1_ref, deg_ref, hs_ref, self_ref, dinv_ref):
    dinv = jax.lax.rsqrt(deg_ref[...])
    h1 = jnp.dot(x_ref[...], w1_ref[...], preferred_element_type=jnp.float32)
    hs_ref[...] = h1 * dinv
    self_ref[...] = h1 * (dinv * dinv)
    dinv_ref[...] = dinv


def _tc2_body(acc_ref, dinv_ref, self_ref, g_ref, bt_ref, w2_ref,
              hs2_ref, self2_ref):
    dinv = dinv_ref[...]
    out1 = (acc_ref[0, :N_NODES] + acc_ref[1, :N_NODES]) * dinv + self_ref[...]
    mean = jnp.mean(out1, axis=0, keepdims=True)
    xc = out1 - mean
    var = jnp.mean(xc * xc, axis=0, keepdims=True)
    y = jnp.maximum(xc * jax.lax.rsqrt(var + EPS) * g_ref[...] + bt_ref[...], 0.0)
    h2 = jnp.dot(y, w2_ref[...], preferred_element_type=jnp.float32)
    hs2_ref[...] = jnp.concatenate(
        [h2 * dinv, jnp.zeros_like(h2)], axis=1)
    self2_ref[...] = h2 * (dinv * dinv)


def _tc3_body(acc_ref, dinv_ref, self2_ref, b2_ref, out_ref):
    acc = acc_ref[0, :N_NODES, :D_OUT] + acc_ref[1, :N_NODES, :D_OUT]
    out_ref[...] = acc * dinv_ref[...] + self2_ref[...] + b2_ref[...]


def kernel(x, edge_index, weight, W1, b1, g1, bt1, W2, b2):
    src = edge_index[0].astype(jnp.int32)
    dst = edge_index[1].astype(jnp.int32)
    w = weight
    f32 = jnp.float32

    zeros640 = jnp.zeros((640,), f32)
    zeros128 = jnp.zeros((ROWS_PT, D_H), f32)

    degp = _deg(dst, w, zeros640)
    deg_col = (degp[0, :N_NODES] + degp[1, :N_NODES] + 1.0).reshape(N_NODES, 1)

    hs1, self1, dinv = pl.pallas_call(
        _tc1_body,
        out_shape=(
            jax.ShapeDtypeStruct((N_NODES, D_H), f32),
            jax.ShapeDtypeStruct((N_NODES, D_H), f32),
            jax.ShapeDtypeStruct((N_NODES, 1), f32),
        ),
    )(x, W1, deg_col)

    acc1p = _msg128(hs1, src, dst, w, zeros128)

    hs2, self2 = pl.pallas_call(
        _tc2_body,
        out_shape=(
            jax.ShapeDtypeStruct((N_NODES, D_H), f32),
            jax.ShapeDtypeStruct((N_NODES, D_OUT), f32),
        ),
    )(acc1p, dinv, self1, g1.reshape(1, D_H), bt1.reshape(1, D_H), W2)

    acc2p = _msg128(hs2, src, dst, w, zeros128)

    out = pl.pallas_call(
        _tc3_body,
        out_shape=jax.ShapeDtypeStruct((N_NODES, D_OUT), f32),
    )(acc2p, dinv, self2, b2.reshape(1, D_OUT))
    return out


# speedup vs baseline: 20.1363x; 2.0614x over previous
import functools

import jax
import jax.numpy as jnp
from jax import lax
from jax.experimental import pallas as pl
from jax.experimental.pallas import tpu as pltpu
from jax.experimental.pallas import tpu_sc as plsc

N_NODES = 10000
N_EDGES = 320000
D_IN = 128
D_H = 128
D_OUT = 64
EPS = 1e-5

NC = 2
NS = 16
NW = NC * NS
NP = 10240
ROWS_PT = NP // NS

CH = 128
K_DEG = 6
N_BIG = 4
BIG = 79 * CH
SMALL = 78 * CH


def _tile_edge_base(wid):
    return jnp.where(wid < N_BIG, wid * BIG, N_BIG * BIG + (wid - N_BIG) * SMALL)


def _make_deg_kernel():
    mesh = plsc.VectorSubcoreMesh(core_axis_name="c", subcore_axis_name="s")

    K = K_DEG
    FULL_SC = 78 // K

    @functools.partial(
        pl.kernel,
        out_type=jax.ShapeDtypeStruct((NC, NP), jnp.float32),
        mesh=mesh,
        scratch_types=(
            [pltpu.VMEM((CH,), jnp.int32) for _ in range(K)]
            + [pltpu.VMEM((CH,), jnp.float32) for _ in range(K)]
            + [pltpu.VMEM_SHARED((NP,), jnp.float32),
               pltpu.SemaphoreType.DMA, pltpu.SemaphoreType.DMA]
        ),
    )
    def deg_kernel(dst_hbm, w_hbm, zeros_hbm, out_hbm, *scr):
        dstv = scr[0:K]
        wv = scr[K:2 * K]
        deg_sp, sem_i, sem_s = scr[2 * K:2 * K + 3]
        cid = lax.axis_index("c")
        sid = lax.axis_index("s")
        pltpu.sync_copy(zeros_hbm, deg_sp.at[pl.ds(sid * 640, 640)])
        plsc.subcore_barrier()
        wid = sid * NC + cid
        base0 = _tile_edge_base(wid)

        def super_chunk(base_sc, nk):
            descs = []
            for k in range(nk):
                b = base_sc + k * CH
                descs.append(pltpu.async_copy(dst_hbm.at[pl.ds(b, CH)], dstv[k], sem_i))
                descs.append(pltpu.async_copy(w_hbm.at[pl.ds(b, CH)], wv[k], sem_i))
            for d in descs:
                d.wait()
            sd = [pltpu.async_copy(wv[k], deg_sp.at[dstv[k]], sem_s, add=True)
                  for k in range(nk)]
            for d in sd:
                d.wait()

        def body(j, carry):
            super_chunk(base0 + j * K * CH, K)
            return carry

        lax.fori_loop(0, FULL_SC, body, 0)

        @pl.when(wid < N_BIG)
        def _tail():
            super_chunk(base0 + FULL_SC * K * CH, 1)

        plsc.subcore_barrier()
        pltpu.sync_copy(deg_sp.at[pl.ds(sid * 640, 640)],
                        out_hbm.at[cid, pl.ds(sid * 640, 640)])

    return deg_kernel


def _make_msg_kernel(D, K):
    FULL_SC = 78 // K
    mesh = plsc.VectorSubcoreMesh(core_axis_name="c", subcore_axis_name="s")

    @functools.partial(
        pl.kernel,
        out_type=jax.ShapeDtypeStruct((NC, NP, D), jnp.float32),
        mesh=mesh,
        scratch_types=(
            [pltpu.VMEM((CH,), jnp.int32) for _ in range(2 * K)]
            + [pltpu.VMEM((CH,), jnp.float32) for _ in range(K)]
            + [pltpu.VMEM((CH, D), jnp.float32) for _ in range(K)]
            + [pltpu.VMEM_SHARED((NP, D), jnp.float32),
               pltpu.SemaphoreType.DMA, pltpu.SemaphoreType.DMA]
            + [pltpu.SemaphoreType.DMA for _ in range(K)]
        ),
    )
    def msg_kernel(hs_hbm, src_hbm, dst_hbm, w_hbm, zeros_hbm, out_hbm, *scr):
        srcv = scr[0:K]
        dstv = scr[K:2 * K]
        wv = scr[2 * K:3 * K]
        rows = scr[3 * K:4 * K]
        acc_sp, sem_i, sem_s = scr[4 * K:4 * K + 3]
        sem_g = scr[4 * K + 3:5 * K + 3]
        cid = lax.axis_index("c")
        sid = lax.axis_index("s")
        pltpu.sync_copy(zeros_hbm, acc_sp.at[pl.ds(sid * ROWS_PT, ROWS_PT)])
        plsc.subcore_barrier()
        wid = sid * NC + cid
        base0 = _tile_edge_base(wid)

        def scale(rows_ref, w_ref):
            def g_body(g, carry):
                wv16 = w_ref[pl.ds(g * 16, 16)]
                for r in range(16):
                    splat = lax.broadcast_in_dim(wv16[r], (16,), ())
                    row = g * 16 + r
                    for c2 in range(D // 16):
                        sl = pl.ds(c2 * 16, 16)
                        rows_ref[row, sl] = rows_ref[row, sl] * splat
                return carry

            lax.fori_loop(0, CH // 16, g_body, 0)

        def super_chunk(base_sc, nk):
            descs = []
            for k in range(nk):
                b = base_sc + k * CH
                descs.append(pltpu.async_copy(src_hbm.at[pl.ds(b, CH)], srcv[k], sem_i))
                descs.append(pltpu.async_copy(dst_hbm.at[pl.ds(b, CH)], dstv[k], sem_i))
                descs.append(pltpu.async_copy(w_hbm.at[pl.ds(b, CH)], wv[k], sem_i))
            for d in descs:
                d.wait()
            gd = [pltpu.async_copy(hs_hbm.at[srcv[k]], rows[k], sem_g[k])
                  for k in range(nk)]
            sd = []
            for k in range(nk):
                gd[k].wait()
                scale(rows[k], wv[k])
                sd.append(pltpu.async_copy(rows[k], acc_sp.at[dstv[k]], sem_s, add=True))
            for d in sd:
                d.wait()

        def body(j, carry):
            super_chunk(base0 + j * K * CH, K)
            return carry

        lax.fori_loop(0, FULL_SC, body, 0)

        @pl.when(wid < N_BIG)
        def _tail():
            super_chunk(base0 + FULL_SC * K * CH, 1)

        plsc.subcore_barrier()
        r0 = sid * ROWS_PT
        pltpu.sync_copy(acc_sp.at[pl.ds(r0, ROWS_PT)],
                        out_hbm.at[cid, pl.ds(r0, ROWS_PT)])

    return msg_kernel


_deg = _make_deg_kernel()
_msg128 = _make_msg_kernel(D_H, 2)


def _tc1_body(x_ref, w1_ref, deg_ref, hs_ref, self_ref, dinv_ref):
    dinv = jax.lax.rsqrt(deg_ref[...])
    h1 = jnp.dot(x_ref[...], w1_ref[...], preferred_element_type=jnp.float32)
    hs_ref[...] = h1 * dinv
    self_ref[...] = h1 * (dinv * dinv)
    dinv_ref[...] = dinv


def _tc2_body(acc_ref, dinv_ref, self_ref, g_ref, bt_ref, w2_ref,
              hs2_ref, self2_ref):
    dinv = dinv_ref[...]
    out1 = (acc_ref[0, :N_NODES] + acc_ref[1, :N_NODES]) * dinv + self_ref[...]
    mean = jnp.mean(out1, axis=0, keepdims=True)
    xc = out1 - mean
    var = jnp.mean(xc * xc, axis=0, keepdims=True)
    y = jnp.maximum(xc * jax.lax.rsqrt(var + EPS) * g_ref[...] + bt_ref[...], 0.0)
    h2 = jnp.dot(y, w2_ref[...], preferred_element_type=jnp.float32)
    hs2_ref[...] = jnp.concatenate([h2 * dinv, jnp.zeros_like(h2)], axis=1)
    self2_ref[...] = h2 * (dinv * dinv)


def _tc3_body(acc_ref, dinv_ref, self2_ref, b2_ref, out_ref):
    acc = acc_ref[0, :N_NODES, :D_OUT] + acc_ref[1, :N_NODES, :D_OUT]
    out_ref[...] = acc * dinv_ref[...] + self2_ref[...] + b2_ref[...]


def kernel(x, edge_index, weight, W1, b1, g1, bt1, W2, b2):
    src = edge_index[0].astype(jnp.int32)
    dst = edge_index[1].astype(jnp.int32)
    w = weight
    f32 = jnp.float32

    zeros640 = jnp.zeros((640,), f32)
    zeros128 = jnp.zeros((ROWS_PT, D_H), f32)

    degp = _deg(dst, w, zeros640)
    deg_col = (degp[0, :N_NODES] + degp[1, :N_NODES] + 1.0).reshape(N_NODES, 1)

    hs1, self1, dinv = pl.pallas_call(
        _tc1_body,
        out_shape=(
            jax.ShapeDtypeStruct((N_NODES, D_H), f32),
            jax.ShapeDtypeStruct((N_NODES, D_H), f32),
            jax.ShapeDtypeStruct((N_NODES, 1), f32),
        ),
    )(x, W1, deg_col)

    acc1p = _msg128(hs1, src, dst, w, zeros128)

    hs2, self2 = pl.pallas_call(
        _tc2_body,
        out_shape=(
            jax.ShapeDtypeStruct((N_NODES, D_H), f32),
            jax.ShapeDtypeStruct((N_NODES, D_OUT), f32),
        ),
    )(acc1p, dinv, self1, g1.reshape(1, D_H), bt1.reshape(1, D_H), W2)

    acc2p = _msg128(hs2, src, dst, w, zeros128)

    out = pl.pallas_call(
        _tc3_body,
        out_shape=jax.ShapeDtypeStruct((N_NODES, D_OUT), f32),
    )(acc2p, dinv, self2, b2.reshape(1, D_OUT))
    return out


# speedup vs baseline: 20.7090x; 1.0284x over previous
import functools

import jax
import jax.numpy as jnp
from jax import lax
from jax.experimental import pallas as pl
from jax.experimental.pallas import tpu as pltpu
from jax.experimental.pallas import tpu_sc as plsc

N_NODES = 10000
N_EDGES = 320000
D_IN = 128
D_H = 128
D_OUT = 64
EPS = 1e-5

NC = 2
NS = 16
NW = NC * NS
NP = 10240
ROWS_PT = NP // NS

CH = 128
K_DEG = 6
N_BIG = 4
BIG = 79 * CH
SMALL = 78 * CH


def _tile_edge_base(wid):
    return jnp.where(wid < N_BIG, wid * BIG, N_BIG * BIG + (wid - N_BIG) * SMALL)


def _make_deg_kernel():
    mesh = plsc.VectorSubcoreMesh(core_axis_name="c", subcore_axis_name="s")

    K = K_DEG
    FULL_SC = 78 // K

    @functools.partial(
        pl.kernel,
        out_type=jax.ShapeDtypeStruct((NC, NP), jnp.float32),
        mesh=mesh,
        scratch_types=(
            [pltpu.VMEM((CH,), jnp.int32) for _ in range(K)]
            + [pltpu.VMEM((CH,), jnp.float32) for _ in range(K)]
            + [pltpu.VMEM_SHARED((NP,), jnp.float32),
               pltpu.SemaphoreType.DMA, pltpu.SemaphoreType.DMA]
        ),
    )
    def deg_kernel(dst_hbm, w_hbm, zeros_hbm, out_hbm, *scr):
        dstv = scr[0:K]
        wv = scr[K:2 * K]
        deg_sp, sem_i, sem_s = scr[2 * K:2 * K + 3]
        cid = lax.axis_index("c")
        sid = lax.axis_index("s")
        pltpu.sync_copy(zeros_hbm, deg_sp.at[pl.ds(sid * 640, 640)])
        plsc.subcore_barrier()
        wid = sid * NC + cid
        base0 = _tile_edge_base(wid)

        def super_chunk(base_sc, nk):
            descs = []
            for k in range(nk):
                b = base_sc + k * CH
                descs.append(pltpu.async_copy(dst_hbm.at[pl.ds(b, CH)], dstv[k], sem_i))
                descs.append(pltpu.async_copy(w_hbm.at[pl.ds(b, CH)], wv[k], sem_i))
            for d in descs:
                d.wait()
            sd = [pltpu.async_copy(wv[k], deg_sp.at[dstv[k]], sem_s, add=True)
                  for k in range(nk)]
            for d in sd:
                d.wait()

        def body(j, carry):
            super_chunk(base0 + j * K * CH, K)
            return carry

        lax.fori_loop(0, FULL_SC, body, 0)

        @pl.when(wid < N_BIG)
        def _tail():
            super_chunk(base0 + FULL_SC * K * CH, 1)

        plsc.subcore_barrier()
        pltpu.sync_copy(deg_sp.at[pl.ds(sid * 640, 640)],
                        out_hbm.at[cid, pl.ds(sid * 640, 640)])

    return deg_kernel


def _make_msg_kernel(D, K, DS):
    FULL_SC = 78 // K
    mesh = plsc.VectorSubcoreMesh(core_axis_name="c", subcore_axis_name="s")

    @functools.partial(
        pl.kernel,
        out_type=jax.ShapeDtypeStruct((NC, NP, D), jnp.float32),
        mesh=mesh,
        scratch_types=(
            [pltpu.VMEM((CH,), jnp.int32) for _ in range(2 * K)]
            + [pltpu.VMEM((CH,), jnp.float32) for _ in range(K)]
            + [pltpu.VMEM((CH, D), jnp.float32) for _ in range(K)]
            + [pltpu.VMEM_SHARED((NP, D), jnp.float32),
               pltpu.SemaphoreType.DMA, pltpu.SemaphoreType.DMA]
            + [pltpu.SemaphoreType.DMA for _ in range(K)]
        ),
    )
    def msg_kernel(hs_hbm, src_hbm, dst_hbm, w_hbm, zeros_hbm, out_hbm, *scr):
        srcv = scr[0:K]
        dstv = scr[K:2 * K]
        wv = scr[2 * K:3 * K]
        rows = scr[3 * K:4 * K]
        acc_sp, sem_i, sem_s = scr[4 * K:4 * K + 3]
        sem_g = scr[4 * K + 3:5 * K + 3]
        cid = lax.axis_index("c")
        sid = lax.axis_index("s")
        pltpu.sync_copy(zeros_hbm, acc_sp.at[pl.ds(sid * ROWS_PT, ROWS_PT)])
        plsc.subcore_barrier()
        wid = sid * NC + cid
        base0 = _tile_edge_base(wid)

        def scale(rows_ref, w_ref):
            def g_body(g, carry):
                wv16 = w_ref[pl.ds(g * 16, 16)]
                for r in range(16):
                    splat = lax.broadcast_in_dim(wv16[r], (16,), ())
                    row = g * 16 + r
                    for c2 in range(DS // 16):
                        sl = pl.ds(c2 * 16, 16)
                        rows_ref[row, sl] = rows_ref[row, sl] * splat
                return carry

            lax.fori_loop(0, CH // 16, g_body, 0)

        def super_chunk(base_sc, nk):
            descs = []
            for k in range(nk):
                b = base_sc + k * CH
                descs.append(pltpu.async_copy(src_hbm.at[pl.ds(b, CH)], srcv[k], sem_i))
                descs.append(pltpu.async_copy(dst_hbm.at[pl.ds(b, CH)], dstv[k], sem_i))
                descs.append(pltpu.async_copy(w_hbm.at[pl.ds(b, CH)], wv[k], sem_i))
            for d in descs:
                d.wait()
            gd = [pltpu.async_copy(hs_hbm.at[srcv[k]], rows[k], sem_g[k])
                  for k in range(nk)]
            sd = []
            for k in range(nk):
                gd[k].wait()
                scale(rows[k], wv[k])
                sd.append(pltpu.async_copy(rows[k], acc_sp.at[dstv[k]], sem_s, add=True))
            for d in sd:
                d.wait()

        def body(j, carry):
            super_chunk(base0 + j * K * CH, K)
            return carry

        lax.fori_loop(0, FULL_SC, body, 0)

        @pl.when(wid < N_BIG)
        def _tail():
            super_chunk(base0 + FULL_SC * K * CH, 1)

        plsc.subcore_barrier()
        r0 = sid * ROWS_PT
        pltpu.sync_copy(acc_sp.at[pl.ds(r0, ROWS_PT)],
                        out_hbm.at[cid, pl.ds(r0, ROWS_PT)])

    return msg_kernel


_deg = _make_deg_kernel()
_msg128 = _make_msg_kernel(D_H, 2, D_H)
_msg128h = _make_msg_kernel(D_H, 2, D_OUT)


def _mm1_body(x_ref, w1_ref, h1_ref):
    h1_ref[...] = jnp.dot(x_ref[...], w1_ref[...],
                          preferred_element_type=jnp.float32)


def _tc1_body(h1_ref, deg_ref, hs_ref, self_ref, dinv_ref):
    dinv = jax.lax.rsqrt(deg_ref[...])
    h1 = h1_ref[...]
    hs_ref[...] = h1 * dinv
    self_ref[...] = h1 * (dinv * dinv)
    dinv_ref[...] = dinv


def _tc2_body(acc_ref, dinv_ref, self_ref, g_ref, bt_ref, w2_ref,
              hs2_ref, self2_ref):
    dinv = dinv_ref[...]
    out1 = (acc_ref[0, :N_NODES] + acc_ref[1, :N_NODES]) * dinv + self_ref[...]
    mean = jnp.mean(out1, axis=0, keepdims=True)
    xc = out1 - mean
    var = jnp.mean(xc * xc, axis=0, keepdims=True)
    y = jnp.maximum(xc * jax.lax.rsqrt(var + EPS) * g_ref[...] + bt_ref[...], 0.0)
    h2 = jnp.dot(y, w2_ref[...], preferred_element_type=jnp.float32)
    hs2_ref[...] = jnp.concatenate([h2 * dinv, jnp.zeros_like(h2)], axis=1)
    self2_ref[...] = h2 * (dinv * dinv)


def _tc3_body(acc_ref, dinv_ref, self2_ref, b2_ref, out_ref):
    acc = acc_ref[0, :N_NODES, :D_OUT] + acc_ref[1, :N_NODES, :D_OUT]
    out_ref[...] = acc * dinv_ref[...] + self2_ref[...] + b2_ref[...]


def kernel(x, edge_index, weight, W1, b1, g1, bt1, W2, b2):
    src = edge_index[0].astype(jnp.int32)
    dst = edge_index[1].astype(jnp.int32)
    w = weight
    f32 = jnp.float32

    zeros640 = jnp.zeros((640,), f32)
    zeros128 = jnp.zeros((ROWS_PT, D_H), f32)

    degp = _deg(dst, w, zeros640)
    h1 = pl.pallas_call(
        _mm1_body,
        out_shape=jax.ShapeDtypeStruct((N_NODES, D_H), f32),
    )(x, W1)
    deg_col = (degp[0, :N_NODES] + degp[1, :N_NODES] + 1.0).reshape(N_NODES, 1)

    hs1, self1, dinv = pl.pallas_call(
        _tc1_body,
        out_shape=(
            jax.ShapeDtypeStruct((N_NODES, D_H), f32),
            jax.ShapeDtypeStruct((N_NODES, D_H), f32),
            jax.ShapeDtypeStruct((N_NODES, 1), f32),
        ),
    )(h1, deg_col)

    acc1p = _msg128(hs1, src, dst, w, zeros128)

    hs2, self2 = pl.pallas_call(
        _tc2_body,
        out_shape=(
            jax.ShapeDtypeStruct((N_NODES, D_H), f32),
            jax.ShapeDtypeStruct((N_NODES, D_OUT), f32),
        ),
    )(acc1p, dinv, self1, g1.reshape(1, D_H), bt1.reshape(1, D_H), W2)

    acc2p = _msg128h(hs2, src, dst, w, zeros128)

    out = pl.pallas_call(
        _tc3_body,
        out_shape=jax.ShapeDtypeStruct((N_NODES, D_OUT), f32),
    )(acc2p, dinv, self2, b2.reshape(1, D_OUT))
    return out


# speedup vs baseline: 21.3653x; 1.0317x over previous
import functools

import jax
import jax.numpy as jnp
from jax import lax
from jax.experimental import pallas as pl
from jax.experimental.pallas import tpu as pltpu
from jax.experimental.pallas import tpu_sc as plsc

N_NODES = 10000
N_EDGES = 320000
D_IN = 128
D_H = 128
D_OUT = 64
EPS = 1e-5

NC = 2
NS = 16
NW = NC * NS
NP = 10240
NP2 = 10112
ROWS_PT = NP2 // NS

CH = 128
K_DEG = 6
N_BIG = 4
BIG = 79 * CH
SMALL = 78 * CH


def _tile_edge_base(wid):
    return jnp.where(wid < N_BIG, wid * BIG, N_BIG * BIG + (wid - N_BIG) * SMALL)


def _make_deg_kernel():
    mesh = plsc.VectorSubcoreMesh(core_axis_name="c", subcore_axis_name="s")

    K = K_DEG
    FULL_SC = 78 // K

    @functools.partial(
        pl.kernel,
        out_type=jax.ShapeDtypeStruct((NC, NP), jnp.float32),
        mesh=mesh,
        scratch_types=(
            [pltpu.VMEM((CH,), jnp.int32) for _ in range(K)]
            + [pltpu.VMEM((CH,), jnp.float32) for _ in range(K)]
            + [pltpu.VMEM_SHARED((NP,), jnp.float32),
               pltpu.SemaphoreType.DMA, pltpu.SemaphoreType.DMA]
        ),
    )
    def deg_kernel(dst_hbm, w_hbm, zeros_hbm, out_hbm, *scr):
        dstv = scr[0:K]
        wv = scr[K:2 * K]
        deg_sp, sem_i, sem_s = scr[2 * K:2 * K + 3]
        cid = lax.axis_index("c")
        sid = lax.axis_index("s")
        pltpu.sync_copy(zeros_hbm, deg_sp.at[pl.ds(sid * 640, 640)])
        plsc.subcore_barrier()
        wid = sid * NC + cid
        base0 = _tile_edge_base(wid)

        def super_chunk(base_sc, nk):
            descs = []
            for k in range(nk):
                b = base_sc + k * CH
                descs.append(pltpu.async_copy(dst_hbm.at[pl.ds(b, CH)], dstv[k], sem_i))
                descs.append(pltpu.async_copy(w_hbm.at[pl.ds(b, CH)], wv[k], sem_i))
            for d in descs:
                d.wait()
            sd = [pltpu.async_copy(wv[k], deg_sp.at[dstv[k]], sem_s, add=True)
                  for k in range(nk)]
            for d in sd:
                d.wait()

        def body(j, carry):
            super_chunk(base0 + j * K * CH, K)
            return carry

        lax.fori_loop(0, FULL_SC, body, 0)

        @pl.when(wid < N_BIG)
        def _tail():
            super_chunk(base0 + FULL_SC * K * CH, 1)

        plsc.subcore_barrier()
        pltpu.sync_copy(deg_sp.at[pl.ds(sid * 640, 640)],
                        out_hbm.at[cid, pl.ds(sid * 640, 640)])

    return deg_kernel


def _make_msg_kernel(DG, DO, CH_M, K, n_chunks_pt):
    FULL = n_chunks_pt // K
    TAIL = n_chunks_pt - FULL * K
    COMPACT = DO < DG
    mesh = plsc.VectorSubcoreMesh(core_axis_name="c", subcore_axis_name="s")

    scratch = (
        [pltpu.VMEM((CH_M,), jnp.int32) for _ in range(2 * K)]
        + [pltpu.VMEM((CH_M,), jnp.float32) for _ in range(K)]
        + [pltpu.VMEM((CH_M, DG), jnp.float32) for _ in range(K)]
    )
    if COMPACT:
        scratch += [pltpu.VMEM((CH_M, DO), jnp.float32) for _ in range(K)]
    scratch += (
        [pltpu.VMEM_SHARED((NP2, DO), jnp.float32),
         pltpu.SemaphoreType.DMA, pltpu.SemaphoreType.DMA]
        + [pltpu.SemaphoreType.DMA for _ in range(K)]
    )

    @functools.partial(
        pl.kernel,
        out_type=jax.ShapeDtypeStruct((NC, NP2, DO), jnp.float32),
        mesh=mesh,
        scratch_types=tuple(scratch),
    )
    def msg_kernel(hs_hbm, src_hbm, dst_hbm, w_hbm, zeros_hbm, out_hbm, *scr):
        srcv = scr[0:K]
        dstv = scr[K:2 * K]
        wv = scr[2 * K:3 * K]
        rows = scr[3 * K:4 * K]
        p = 4 * K
        if COMPACT:
            outs = scr[p:p + K]
            p += K
        else:
            outs = rows
        acc_sp, sem_i, sem_s = scr[p:p + 3]
        sem_g = scr[p + 3:p + 3 + K]
        cid = lax.axis_index("c")
        sid = lax.axis_index("s")
        pltpu.sync_copy(zeros_hbm, acc_sp.at[pl.ds(sid * ROWS_PT, ROWS_PT)])
        plsc.subcore_barrier()
        wid = sid * NC + cid
        base0 = wid * (n_chunks_pt * CH_M)

        def scale(rows_ref, out_ref, w_ref):
            def g_body(g, carry):
                wv16 = w_ref[pl.ds(g * 16, 16)]
                for r in range(16):
                    splat = lax.broadcast_in_dim(wv16[r], (16,), ())
                    row = g * 16 + r
                    for c2 in range(DO // 16):
                        sl = pl.ds(c2 * 16, 16)
                        out_ref[row, sl] = rows_ref[row, sl] * splat
                return carry

            lax.fori_loop(0, CH_M // 16, g_body, 0)

        def super_chunk(base_sc, nk):
            descs = []
            for k in range(nk):
                b = base_sc + k * CH_M
                descs.append(pltpu.async_copy(src_hbm.at[pl.ds(b, CH_M)], srcv[k], sem_i))
                descs.append(pltpu.async_copy(dst_hbm.at[pl.ds(b, CH_M)], dstv[k], sem_i))
                descs.append(pltpu.async_copy(w_hbm.at[pl.ds(b, CH_M)], wv[k], sem_i))
            for d in descs:
                d.wait()
            gd = [pltpu.async_copy(hs_hbm.at[srcv[k]], rows[k], sem_g[k])
                  for k in range(nk)]
            sd = []
            for k in range(nk):
                gd[k].wait()
                scale(rows[k], outs[k], wv[k])
                sd.append(pltpu.async_copy(outs[k], acc_sp.at[dstv[k]], sem_s, add=True))
            for d in sd:
                d.wait()

        def body(j, carry):
            super_chunk(base0 + j * K * CH_M, K)
            return carry

        lax.fori_loop(0, FULL, body, 0)
        if TAIL:
            super_chunk(base0 + FULL * K * CH_M, TAIL)

        plsc.subcore_barrier()
        r0 = sid * ROWS_PT
        pltpu.sync_copy(acc_sp.at[pl.ds(r0, ROWS_PT)],
                        out_hbm.at[cid, pl.ds(r0, ROWS_PT)])

    return msg_kernel


_deg = _make_deg_kernel()
_msg1 = _make_msg_kernel(D_H, D_H, 80, 4, 125)
_msg2 = _make_msg_kernel(D_H, D_H, 80, 4, 125)


def _mm1_body(x_ref, w1_ref, h1_ref):
    h1_ref[...] = jnp.dot(x_ref[...], w1_ref[...],
                          preferred_element_type=jnp.float32)


def _tc1_body(h1_ref, deg_ref, hs_ref, self_ref, dinv_ref):
    dinv = jax.lax.rsqrt(deg_ref[...])
    h1 = h1_ref[...]
    hs_ref[...] = h1 * dinv
    self_ref[...] = h1 * (dinv * dinv)
    dinv_ref[...] = dinv


def _tc2_body(acc_ref, dinv_ref, self_ref, g_ref, bt_ref, w2_ref,
              hs2_ref, self2_ref):
    dinv = dinv_ref[...]
    out1 = (acc_ref[0, :N_NODES] + acc_ref[1, :N_NODES]) * dinv + self_ref[...]
    mean = jnp.mean(out1, axis=0, keepdims=True)
    xc = out1 - mean
    var = jnp.mean(xc * xc, axis=0, keepdims=True)
    y = jnp.maximum(xc * jax.lax.rsqrt(var + EPS) * g_ref[...] + bt_ref[...], 0.0)
    h2 = jnp.dot(y, w2_ref[...], preferred_element_type=jnp.float32)
    hs2_ref[...] = jnp.concatenate([h2 * dinv, jnp.zeros_like(h2)], axis=1)
    self2_ref[...] = h2 * (dinv * dinv)


def _tc3_body(acc_ref, dinv_ref, self2_ref, b2_ref, out_ref):
    acc = acc_ref[0, :N_NODES, :D_OUT] + acc_ref[1, :N_NODES, :D_OUT]
    out_ref[...] = acc * dinv_ref[...] + self2_ref[...] + b2_ref[...]


def kernel(x, edge_index, weight, W1, b1, g1, bt1, W2, b2):
    src = edge_index[0].astype(jnp.int32)
    dst = edge_index[1].astype(jnp.int32)
    w = weight
    f32 = jnp.float32

    zeros640 = jnp.zeros((640,), f32)
    zeros128 = jnp.zeros((ROWS_PT, D_H), f32)

    degp = _deg(dst, w, zeros640)
    h1 = pl.pallas_call(
        _mm1_body,
        out_shape=jax.ShapeDtypeStruct((N_NODES, D_H), f32),
    )(x, W1)
    deg_col = (degp[0, :N_NODES] + degp[1, :N_NODES] + 1.0).reshape(N_NODES, 1)

    hs1, self1, dinv = pl.pallas_call(
        _tc1_body,
        out_shape=(
            jax.ShapeDtypeStruct((N_NODES, D_H), f32),
            jax.ShapeDtypeStruct((N_NODES, D_H), f32),
            jax.ShapeDtypeStruct((N_NODES, 1), f32),
        ),
    )(h1, deg_col)

    acc1p = _msg1(hs1, src, dst, w, zeros128)

    hs2, self2 = pl.pallas_call(
        _tc2_body,
        out_shape=(
            jax.ShapeDtypeStruct((N_NODES, D_H), f32),
            jax.ShapeDtypeStruct((N_NODES, D_OUT), f32),
        ),
    )(acc1p, dinv, self1, g1.reshape(1, D_H), bt1.reshape(1, D_H), W2)

    acc2p = _msg2(hs2, src, dst, w, zeros128)

    out = pl.pallas_call(
        _tc3_body,
        out_shape=jax.ShapeDtypeStruct((N_NODES, D_OUT), f32),
    )(acc2p, dinv, self2, b2.reshape(1, D_OUT))
    return out


# speedup vs baseline: 22.3498x; 1.0461x over previous
import functools

import jax
import jax.numpy as jnp
from jax import lax
from jax.experimental import pallas as pl
from jax.experimental.pallas import tpu as pltpu
from jax.experimental.pallas import tpu_sc as plsc

N_NODES = 10000
N_EDGES = 320000
D_IN = 128
D_H = 128
D_OUT = 64
EPS = 1e-5

NC = 2
NS = 16
NW = NC * NS
NP = 10240
NP2 = 10112
ROWS_PT = NP2 // NS

CH = 128
K_DEG = 6
N_BIG = 4
BIG = 79 * CH
SMALL = 78 * CH


def _tile_edge_base(wid):
    return jnp.where(wid < N_BIG, wid * BIG, N_BIG * BIG + (wid - N_BIG) * SMALL)


def _make_deg_kernel():
    mesh = plsc.VectorSubcoreMesh(core_axis_name="c", subcore_axis_name="s")

    K = K_DEG
    FULL_SC = 78 // K

    @functools.partial(
        pl.kernel,
        out_type=jax.ShapeDtypeStruct((NC, NP), jnp.float32),
        mesh=mesh,
        scratch_types=(
            [pltpu.VMEM((CH,), jnp.int32) for _ in range(K)]
            + [pltpu.VMEM((CH,), jnp.float32) for _ in range(K)]
            + [pltpu.VMEM_SHARED((NP,), jnp.float32),
               pltpu.SemaphoreType.DMA, pltpu.SemaphoreType.DMA]
        ),
    )
    def deg_kernel(dst_hbm, w_hbm, zeros_hbm, out_hbm, *scr):
        dstv = scr[0:K]
        wv = scr[K:2 * K]
        deg_sp, sem_i, sem_s = scr[2 * K:2 * K + 3]
        cid = lax.axis_index("c")
        sid = lax.axis_index("s")
        pltpu.sync_copy(zeros_hbm, deg_sp.at[pl.ds(sid * 640, 640)])
        plsc.subcore_barrier()
        wid = sid * NC + cid
        base0 = _tile_edge_base(wid)

        def super_chunk(base_sc, nk):
            descs = []
            for k in range(nk):
                b = base_sc + k * CH
                descs.append(pltpu.async_copy(dst_hbm.at[pl.ds(b, CH)], dstv[k], sem_i))
                descs.append(pltpu.async_copy(w_hbm.at[pl.ds(b, CH)], wv[k], sem_i))
            for d in descs:
                d.wait()
            sd = [pltpu.async_copy(wv[k], deg_sp.at[dstv[k]], sem_s, add=True)
                  for k in range(nk)]
            for d in sd:
                d.wait()

        def body(j, carry):
            super_chunk(base0 + j * K * CH, K)
            return carry

        lax.fori_loop(0, FULL_SC, body, 0)

        @pl.when(wid < N_BIG)
        def _tail():
            super_chunk(base0 + FULL_SC * K * CH, 1)

        plsc.subcore_barrier()
        pltpu.sync_copy(deg_sp.at[pl.ds(sid * 640, 640)],
                        out_hbm.at[cid, pl.ds(sid * 640, 640)])

    return deg_kernel


def _make_msg_kernel(DG, DO, CH_M, K, n_chunks_pt):
    FULL = n_chunks_pt // K
    TAIL = n_chunks_pt - FULL * K
    PAIRS = (FULL - 1) // 2
    mesh = plsc.VectorSubcoreMesh(core_axis_name="c", subcore_axis_name="s")

    scratch = (
        [pltpu.VMEM((CH_M,), jnp.int32) for _ in range(4 * K)]
        + [pltpu.VMEM((CH_M,), jnp.float32) for _ in range(2 * K)]
        + [pltpu.VMEM((CH_M, DG), jnp.float32) for _ in range(K)]
        + [pltpu.VMEM_SHARED((NP2, DO), jnp.float32),
           pltpu.SemaphoreType.DMA, pltpu.SemaphoreType.DMA,
           pltpu.SemaphoreType.DMA]
        + [pltpu.SemaphoreType.DMA for _ in range(K)]
    )

    @functools.partial(
        pl.kernel,
        out_type=jax.ShapeDtypeStruct((NC, NP2, DO), jnp.float32),
        mesh=mesh,
        scratch_types=tuple(scratch),
    )
    def msg_kernel(hs_hbm, src_hbm, dst_hbm, w_hbm, zeros_hbm, out_hbm, *scr):
        srcv = (scr[0:K], scr[K:2 * K])
        dstv = (scr[2 * K:3 * K], scr[3 * K:4 * K])
        wv = (scr[4 * K:5 * K], scr[5 * K:6 * K])
        rows = scr[6 * K:7 * K]
        acc_sp, semi0, semi1, sem_s = scr[7 * K:7 * K + 4]
        sem_i = (semi0, semi1)
        sem_g = scr[7 * K + 4:7 * K + 4 + K]
        cid = lax.axis_index("c")
        sid = lax.axis_index("s")
        pltpu.sync_copy(zeros_hbm, acc_sp.at[pl.ds(sid * ROWS_PT, ROWS_PT)])
        plsc.subcore_barrier()
        wid = sid * NC + cid
        base0 = wid * (n_chunks_pt * CH_M)

        def fire_idx(s, sup_base, nk):
            for k in range(nk):
                b = sup_base + k * CH_M
                pltpu.async_copy(src_hbm.at[pl.ds(b, CH_M)], srcv[s][k], sem_i[s])
                pltpu.async_copy(dst_hbm.at[pl.ds(b, CH_M)], dstv[s][k], sem_i[s])
                pltpu.async_copy(w_hbm.at[pl.ds(b, CH_M)], wv[s][k], sem_i[s])

        def drain_idx(s, nk):
            dummy = pl.ds(0, CH_M)
            for k in range(nk):
                pltpu.make_async_copy(src_hbm.at[dummy], srcv[s][k], sem_i[s]).wait()
                pltpu.make_async_copy(dst_hbm.at[dummy], dstv[s][k], sem_i[s]).wait()
                pltpu.make_async_copy(w_hbm.at[dummy], wv[s][k], sem_i[s]).wait()

        def scale(rows_ref, w_ref):
            def g_body(g, carry):
                wv16 = w_ref[pl.ds(g * 16, 16)]
                for r in range(16):
                    splat = lax.broadcast_in_dim(wv16[r], (16,), ())
                    row = g * 16 + r
                    for c2 in range(DO // 16):
                        sl = pl.ds(c2 * 16, 16)
                        rows_ref[row, sl] = rows_ref[row, sl] * splat
                return carry

            lax.fori_loop(0, CH_M // 16, g_body, 0)

        def process(s, nk):
            drain_idx(s, nk)
            gd = [pltpu.async_copy(hs_hbm.at[srcv[s][k]], rows[k], sem_g[k])
                  for k in range(nk)]
            sd = []
            for k in range(nk):
                gd[k].wait()
                scale(rows[k], wv[s][k])
                sd.append(pltpu.async_copy(rows[k], acc_sp.at[dstv[s][k]],
                                           sem_s, add=True))
            for d in sd:
                d.wait()

        fire_idx(0, base0, K)
        fire_idx(1, base0 + K * CH_M, K)

        def body(j, carry):
            sup = 2 * j
            process(0, K)

            @pl.when(j < PAIRS - 1 + (FULL % 2))
            def _pa():
                fire_idx(0, base0 + (sup + 2) * K * CH_M, K)

            process(1, K)

            @pl.when(j < PAIRS - 1)
            def _pb():
                fire_idx(1, base0 + (sup + 3) * K * CH_M, K)

            return carry

        lax.fori_loop(0, PAIRS, body, 0)
        if FULL % 2:
            process(0, K)
        if TAIL:
            fire_idx(FULL % 2, base0 + FULL * K * CH_M, TAIL)
            process(FULL % 2, TAIL)

        plsc.subcore_barrier()
        r0 = sid * ROWS_PT
        pltpu.sync_copy(acc_sp.at[pl.ds(r0, ROWS_PT)],
                        out_hbm.at[cid, pl.ds(r0, ROWS_PT)])

    return msg_kernel


_deg = _make_deg_kernel()
_msg = _make_msg_kernel(D_H, D_H, 80, 4, 125)


def _mm1_body(x_ref, w1_ref, h1_ref):
    h1_ref[...] = jnp.dot(x_ref[...], w1_ref[...],
                          preferred_element_type=jnp.float32)


def _tc1_body(h1_ref, deg_ref, hs_ref, self_ref, dinv_ref):
    dinv = jax.lax.rsqrt(deg_ref[...])
    h1 = h1_ref[...]
    hs_ref[...] = h1 * dinv
    self_ref[...] = h1 * (dinv * dinv)
    dinv_ref[...] = dinv


def _tc2_body(acc_ref, dinv_ref, self_ref, g_ref, bt_ref, w2_ref,
              hs2_ref, self2_ref):
    dinv = dinv_ref[...]
    out1 = (acc_ref[0, :N_NODES] + acc_ref[1, :N_NODES]) * dinv + self_ref[...]
    mean = jnp.mean(out1, axis=0, keepdims=True)
    xc = out1 - mean
    var = jnp.mean(xc * xc, axis=0, keepdims=True)
    y = jnp.maximum(xc * jax.lax.rsqrt(var + EPS) * g_ref[...] + bt_ref[...], 0.0)
    h2 = jnp.dot(y, w2_ref[...], preferred_element_type=jnp.float32)
    hs2_ref[...] = jnp.concatenate([h2 * dinv, jnp.zeros_like(h2)], axis=1)
    self2_ref[...] = h2 * (dinv * dinv)


def _tc3_body(acc_ref, dinv_ref, self2_ref, b2_ref, out_ref):
    acc = acc_ref[0, :N_NODES, :D_OUT] + acc_ref[1, :N_NODES, :D_OUT]
    out_ref[...] = acc * dinv_ref[...] + self2_ref[...] + b2_ref[...]


def kernel(x, edge_index, weight, W1, b1, g1, bt1, W2, b2):
    src = edge_index[0].astype(jnp.int32)
    dst = edge_index[1].astype(jnp.int32)
    w = weight
    f32 = jnp.float32

    zeros640 = jnp.zeros((640,), f32)
    zeros128 = jnp.zeros((ROWS_PT, D_H), f32)

    degp = _deg(dst, w, zeros640)
    h1 = pl.pallas_call(
        _mm1_body,
        out_shape=jax.ShapeDtypeStruct((N_NODES, D_H), f32),
    )(x, W1)
    deg_col = (degp[0, :N_NODES] + degp[1, :N_NODES] + 1.0).reshape(N_NODES, 1)

    hs1, self1, dinv = pl.pallas_call(
        _tc1_body,
        out_shape=(
            jax.ShapeDtypeStruct((N_NODES, D_H), f32),
            jax.ShapeDtypeStruct((N_NODES, D_H), f32),
            jax.ShapeDtypeStruct((N_NODES, 1), f32),
        ),
    )(h1, deg_col)

    acc1p = _msg(hs1, src, dst, w, zeros128)

    hs2, self2 = pl.pallas_call(
        _tc2_body,
        out_shape=(
            jax.ShapeDtypeStruct((N_NODES, D_H), f32),
            jax.ShapeDtypeStruct((N_NODES, D_OUT), f32),
        ),
    )(acc1p, dinv, self1, g1.reshape(1, D_H), bt1.reshape(1, D_H), W2)

    acc2p = _msg(hs2, src, dst, w, zeros128)

    out = pl.pallas_call(
        _tc3_body,
        out_shape=jax.ShapeDtypeStruct((N_NODES, D_OUT), f32),
    )(acc2p, dinv, self2, b2.reshape(1, D_OUT))
    return out
